# baseline (device time: 213257 ns/iter reference)
import jax
import jax.numpy as jnp
from jax import lax
from jax.experimental import pallas as pl
from jax.experimental.pallas import tpu as pltpu

N_DEV = 8
N_SUB = 4
DEPTH = 3
N_HOPS = 2 * (N_DEV - 1)


def _gelu(y):
    c = 0.7978845608028654
    return 0.5 * y * (1.0 + jnp.tanh(c * (y + 0.044715 * y * y * y)))


def kernel(x, w_mat):
    m, k_sh = x.shape
    _, n = w_mat.shape
    ch = m // N_DEV
    chh = ch // N_SUB
    nh = n // 2
    x = x.astype(jnp.bfloat16)
    w_mat = w_mat.astype(jnp.bfloat16)

    def body(x_ref, w_ref, out_ref, comm_r, comm_l, stage_r, stage_l, sems):
        my = lax.axis_index("i")
        left = lax.rem(my + N_DEV - 1, N_DEV)
        right = lax.rem(my + 1, N_DEV)

        barrier_sem = pltpu.get_barrier_semaphore()
        for nbr in (left, right):
            pl.semaphore_signal(
                barrier_sem, inc=1,
                device_id=(nbr,), device_id_type=pl.DeviceIdType.MESH,
            )
        pl.semaphore_wait(barrier_sem, 2)

        def partial_sub(c, half, sub):
            xs = x_ref[pl.ds(c * ch + sub * chh, chh), :]
            ws = w_ref[:, half * nh:(half + 1) * nh]
            return lax.dot_general(
                xs, ws,
                (((1,), (0,)), ((), ())),
                preferred_element_type=jnp.float32,
            )

        comms = (comm_r, comm_l)
        dsts = (right, left)
        descs = {}

        def make(s, direction, sub):
            comm = comms[direction]
            send_slot = s % DEPTH
            recv_slot = (s + 1) % DEPTH
            rows = pl.ds(sub * chh, chh)
            return pltpu.make_async_remote_copy(
                src_ref=comm.at[send_slot, rows],
                dst_ref=comm.at[recv_slot, rows],
                send_sem=sems.at[direction, sub, 0, send_slot],
                recv_sem=sems.at[direction, sub, 1, recv_slot],
                device_id=(dsts[direction],),
                device_id_type=pl.DeviceIdType.MESH,
            )

        def start(s, direction, sub):
            if s >= DEPTH:
                descs[(s - DEPTH, direction, sub)].wait_send()
            d = make(s, direction, sub)
            descs[(s, direction, sub)] = d
            d.start()

        stages = (stage_r, stage_l)

        def acc(direction, recv_slot, stage_slot, sub):
            comm = comms[direction]
            rows = pl.ds(sub * chh, chh)
            comm[recv_slot, rows, :] = (
                comm[recv_slot, rows, :].astype(jnp.float32)
                + stages[direction][stage_slot, rows, :]
            ).astype(jnp.bfloat16)

        for sub in range(N_SUB):
            rows = pl.ds(sub * chh, chh)
            comm_r[0, rows, :] = partial_sub(my, 0, sub).astype(jnp.bfloat16)
            comm_l[0, rows, :] = partial_sub(my, 1, sub).astype(jnp.bfloat16)
            start(0, 0, sub)
            start(0, 1, sub)
        c_dir0 = (lax.rem(my + 2 * N_DEV - 1, N_DEV), lax.rem(my + 1, N_DEV))
        for sub in range(N_SUB):
            for direction in range(2):
                stages[direction][0, pl.ds(sub * chh, chh), :] = partial_sub(
                    c_dir0[direction], direction, sub
                )
        for s in range(N_DEV - 1):
            recv_slot = (s + 1) % DEPTH
            stage_slot = s % 2
            for sub in range(N_SUB):
                for direction in range(2):
                    descs[(s, direction, sub)].wait_recv()
                    acc(direction, recv_slot, stage_slot, sub)
                    if s < N_DEV - 2:
                        start(s + 1, direction, sub)
            if s < N_DEV - 2:
                c_dir = (
                    lax.rem(my + 2 * N_DEV - s - 2, N_DEV),
                    lax.rem(my + s + 2, N_DEV),
                )
                for sub in range(N_SUB):
                    for direction in range(2):
                        stages[direction][
                            1 - stage_slot, pl.ds(sub * chh, chh), :
                        ] = partial_sub(c_dir[direction], direction, sub)

        red_slot = (N_DEV - 1) % DEPTH
        own = (lax.rem(my + 1, N_DEV), lax.rem(my + N_DEV - 1, N_DEV))
        cols = (slice(None, nh), slice(nh, None))
        for sub in range(N_SUB):
            rows = pl.ds(sub * chh, chh)
            for direction in range(2):
                comm = comms[direction]
                ge = _gelu(comm[red_slot, rows, :].astype(jnp.float32))
                out_ref[pl.ds(own[direction] * ch + sub * chh, chh),
                        cols[direction]] = ge
                comm[red_slot, rows, :] = ge.astype(jnp.bfloat16)
                start(N_DEV - 1, direction, sub)

        for t in range(N_DEV - 1):
            s = N_DEV - 1 + t
            recv_slot = (s + 1) % DEPTH
            c_dir = (
                lax.rem(my + 2 * N_DEV - t, N_DEV),
                lax.rem(my + t, N_DEV),
            )
            for sub in range(N_SUB):
                rows = pl.ds(sub * chh, chh)
                for direction in range(2):
                    descs[(s, direction, sub)].wait_recv()
                    if t < N_DEV - 2:
                        start(s + 1, direction, sub)
                    out_ref[
                        pl.ds(c_dir[direction] * ch + sub * chh, chh),
                        cols[direction],
                    ] = comms[direction][recv_slot, rows, :].astype(jnp.float32)

        for s in range(N_HOPS - DEPTH, N_HOPS):
            for sub in range(N_SUB):
                for direction in range(2):
                    descs[(s, direction, sub)].wait_send()

    out_shape = jax.ShapeDtypeStruct((m, n), jnp.float32)
    return pl.pallas_call(
        body,
        out_shape=out_shape,
        in_specs=[
            pl.BlockSpec(memory_space=pltpu.VMEM),
            pl.BlockSpec(memory_space=pltpu.VMEM),
        ],
        out_specs=pl.BlockSpec(memory_space=pltpu.VMEM),
        scratch_shapes=[
            pltpu.VMEM((DEPTH, ch, nh), jnp.bfloat16),
            pltpu.VMEM((DEPTH, ch, nh), jnp.bfloat16),
            pltpu.VMEM((2, ch, nh), jnp.float32),
            pltpu.VMEM((2, ch, nh), jnp.float32),
            pltpu.SemaphoreType.DMA((2, N_SUB, 2, DEPTH)),
        ],
        compiler_params=pltpu.CompilerParams(
            collective_id=0,
            vmem_limit_bytes=60 * 1024 * 1024,
        ),
    )(x, w_mat)


# device time: 192290 ns/iter; 1.1090x vs baseline; 1.1090x over previous
import jax
import jax.numpy as jnp
from jax import lax
from jax.experimental import pallas as pl
from jax.experimental.pallas import tpu as pltpu

N_DEV = 8
N_SUB = 2
DEPTH = 3
N_HOPS = 2 * (N_DEV - 1)


def _gelu(y):
    c = 0.7978845608028654
    return 0.5 * y * (1.0 + jnp.tanh(c * (y + 0.044715 * y * y * y)))


def kernel(x, w_mat):
    m, k_sh = x.shape
    _, n = w_mat.shape
    ch = m // N_DEV
    chh = ch // N_SUB
    nh = n // 2

    def body(x_ref, w_ref, out_ref, comm_r, comm_l, xbf, wbf, sems):
        my = lax.axis_index("i")
        left = lax.rem(my + N_DEV - 1, N_DEV)
        right = lax.rem(my + 1, N_DEV)

        barrier_sem = pltpu.get_barrier_semaphore()
        for nbr in (left, right):
            pl.semaphore_signal(
                barrier_sem, inc=1,
                device_id=(nbr,), device_id_type=pl.DeviceIdType.MESH,
            )
        pl.semaphore_wait(barrier_sem, 2)

        wbf[...] = w_ref[...].astype(jnp.bfloat16)
        xbf[...] = x_ref[...].astype(jnp.bfloat16)

        def partial_sub(c, half, sub):
            xs = xbf[pl.ds(c * ch + sub * chh, chh), :]
            ws = wbf[:, half * nh:(half + 1) * nh]
            return lax.dot_general(
                xs, ws,
                (((1,), (0,)), ((), ())),
                preferred_element_type=jnp.float32,
            )

        comms = (comm_r, comm_l)
        dsts = (right, left)
        cols = (slice(None, nh), slice(nh, None))
        descs = {}

        def sem_pair(s, direction, sub):
            return (
                sems.at[direction, sub, 0, s % DEPTH],
                sems.at[direction, sub, 1, (s + 1) % DEPTH],
            )

        def make_rs(s, direction, sub):
            comm = comms[direction]
            rows = pl.ds(sub * chh, chh)
            send_sem, recv_sem = sem_pair(s, direction, sub)
            return pltpu.make_async_remote_copy(
                src_ref=comm.at[s % DEPTH, rows],
                dst_ref=comm.at[(s + 1) % DEPTH, rows],
                send_sem=send_sem,
                recv_sem=recv_sem,
                device_id=(dsts[direction],),
                device_id_type=pl.DeviceIdType.MESH,
            )

        def make_ag(s, direction, sub, c_send):
            rows = pl.ds(c_send * ch + sub * chh, chh)
            ref = out_ref.at[rows, cols[direction]]
            send_sem, recv_sem = sem_pair(s, direction, sub)
            return pltpu.make_async_remote_copy(
                src_ref=ref,
                dst_ref=ref,
                send_sem=send_sem,
                recv_sem=recv_sem,
                device_id=(dsts[direction],),
                device_id_type=pl.DeviceIdType.MESH,
            )

        def start(s, direction, sub, desc):
            if s >= DEPTH:
                descs[(s - DEPTH, direction, sub)].wait_send()
            descs[(s, direction, sub)] = desc
            desc.start()

        def acc(direction, recv_slot, c, sub):
            comm = comms[direction]
            rows = pl.ds(sub * chh, chh)
            comm[recv_slot, rows, :] = (
                comm[recv_slot, rows, :].astype(jnp.float32)
                + partial_sub(c, direction, sub)
            ).astype(jnp.bfloat16)

        for sub in range(N_SUB):
            rows = pl.ds(sub * chh, chh)
            comm_r[0, rows, :] = partial_sub(my, 0, sub).astype(jnp.bfloat16)
            comm_l[0, rows, :] = partial_sub(my, 1, sub).astype(jnp.bfloat16)
            start(0, 0, sub, make_rs(0, 0, sub))
            start(0, 1, sub, make_rs(0, 1, sub))
        for s in range(N_DEV - 1):
            recv_slot = (s + 1) % DEPTH
            c_dir = (
                lax.rem(my + 2 * N_DEV - s - 1, N_DEV),
                lax.rem(my + s + 1, N_DEV),
            )
            for sub in range(N_SUB):
                for direction in range(2):
                    descs[(s, direction, sub)].wait_recv()
                    acc(direction, recv_slot, c_dir[direction], sub)
                    if s < N_DEV - 2:
                        start(s + 1, direction, sub,
                              make_rs(s + 1, direction, sub))

        red_slot = (N_DEV - 1) % DEPTH
        own = (lax.rem(my + 1, N_DEV), lax.rem(my + N_DEV - 1, N_DEV))
        for sub in range(N_SUB):
            rows = pl.ds(sub * chh, chh)
            for direction in range(2):
                ge = _gelu(comms[direction][red_slot, rows, :].astype(jnp.float32))
                out_ref[pl.ds(own[direction] * ch + sub * chh, chh),
                        cols[direction]] = ge.astype(jnp.bfloat16)
                start(N_DEV - 1, direction, sub,
                      make_ag(N_DEV - 1, direction, sub, own[direction]))

        for t in range(N_DEV - 1):
            s = N_DEV - 1 + t
            for sub in range(N_SUB):
                for direction in range(2):
                    descs[(s, direction, sub)].wait_recv()
                    if t < N_DEV - 2:
                        c_recv = (
                            lax.rem(my + 2 * N_DEV - t, N_DEV),
                            lax.rem(my + t, N_DEV),
                        )[direction]
                        start(s + 1, direction, sub,
                              make_ag(s + 1, direction, sub, c_recv))

        for s in range(N_HOPS - DEPTH, N_HOPS):
            for sub in range(N_SUB):
                for direction in range(2):
                    descs[(s, direction, sub)].wait_send()

    out_shape = jax.ShapeDtypeStruct((m, n), jnp.bfloat16)
    return pl.pallas_call(
        body,
        out_shape=out_shape,
        in_specs=[
            pl.BlockSpec(memory_space=pltpu.VMEM),
            pl.BlockSpec(memory_space=pltpu.VMEM),
        ],
        out_specs=pl.BlockSpec(memory_space=pltpu.VMEM),
        scratch_shapes=[
            pltpu.VMEM((DEPTH, ch, nh), jnp.bfloat16),
            pltpu.VMEM((DEPTH, ch, nh), jnp.bfloat16),
            pltpu.VMEM((m, k_sh), jnp.bfloat16),
            pltpu.VMEM((k_sh, n), jnp.bfloat16),
            pltpu.SemaphoreType.DMA((2, N_SUB, 2, DEPTH)),
        ],
        compiler_params=pltpu.CompilerParams(
            collective_id=0,
            vmem_limit_bytes=60 * 1024 * 1024,
        ),
    )(x, w_mat)


# device time: 192040 ns/iter; 1.1105x vs baseline; 1.0013x over previous
import jax
import jax.numpy as jnp
from jax import lax
from jax.experimental import pallas as pl
from jax.experimental.pallas import tpu as pltpu

N_DEV = 8
N_SUB = 2
DEPTH = 3
N_HOPS = 2 * (N_DEV - 1)


def _gelu(y):
    c = 0.7978845608028654
    return 0.5 * y * (1.0 + jnp.tanh(c * (y + 0.044715 * y * y * y)))


def kernel(x, w_mat):
    m, k_sh = x.shape
    _, n = w_mat.shape
    ch = m // N_DEV
    chh = ch // N_SUB
    nh = n // 2

    def body(x_ref, w_ref, out_ref, comm_r, comm_l, xbf, wbf, sems):
        my = lax.axis_index("i")
        left = lax.rem(my + N_DEV - 1, N_DEV)
        right = lax.rem(my + 1, N_DEV)

        barrier_sem = pltpu.get_barrier_semaphore()
        for nbr in (left, right):
            pl.semaphore_signal(
                barrier_sem, inc=1,
                device_id=(nbr,), device_id_type=pl.DeviceIdType.MESH,
            )
        pl.semaphore_wait(barrier_sem, 2)

        wbf[...] = w_ref[...].astype(jnp.bfloat16)
        own_rows = pl.ds(my * ch, ch)
        xbf[own_rows, :] = x_ref[own_rows, :].astype(jnp.bfloat16)

        def partial_sub(c, half, sub):
            xs = xbf[pl.ds(c * ch + sub * chh, chh), :]
            ws = wbf[:, half * nh:(half + 1) * nh]
            return lax.dot_general(
                xs, ws,
                (((1,), (0,)), ((), ())),
                preferred_element_type=jnp.float32,
            )

        comms = (comm_r, comm_l)
        dsts = (right, left)
        cols = (slice(None, nh), slice(nh, None))
        descs = {}

        def sem_pair(s, direction, sub):
            return (
                sems.at[direction, sub, 0, s % DEPTH],
                sems.at[direction, sub, 1, (s + 1) % DEPTH],
            )

        def make_rs(s, direction, sub):
            comm = comms[direction]
            rows = pl.ds(sub * chh, chh)
            send_sem, recv_sem = sem_pair(s, direction, sub)
            return pltpu.make_async_remote_copy(
                src_ref=comm.at[s % DEPTH, rows],
                dst_ref=comm.at[(s + 1) % DEPTH, rows],
                send_sem=send_sem,
                recv_sem=recv_sem,
                device_id=(dsts[direction],),
                device_id_type=pl.DeviceIdType.MESH,
            )

        def make_ag(s, direction, sub, c_send):
            rows = pl.ds(c_send * ch + sub * chh, chh)
            ref = out_ref.at[rows, cols[direction]]
            send_sem, recv_sem = sem_pair(s, direction, sub)
            return pltpu.make_async_remote_copy(
                src_ref=ref,
                dst_ref=ref,
                send_sem=send_sem,
                recv_sem=recv_sem,
                device_id=(dsts[direction],),
                device_id_type=pl.DeviceIdType.MESH,
            )

        def start(s, direction, sub, desc):
            if s >= DEPTH:
                descs[(s - DEPTH, direction, sub)].wait_send()
            descs[(s, direction, sub)] = desc
            desc.start()

        def acc(direction, recv_slot, c, sub):
            comm = comms[direction]
            rows = pl.ds(sub * chh, chh)
            comm[recv_slot, rows, :] = (
                comm[recv_slot, rows, :].astype(jnp.float32)
                + partial_sub(c, direction, sub)
            ).astype(jnp.bfloat16)

        for sub in range(N_SUB):
            rows = pl.ds(sub * chh, chh)
            comm_r[0, rows, :] = partial_sub(my, 0, sub).astype(jnp.bfloat16)
            comm_l[0, rows, :] = partial_sub(my, 1, sub).astype(jnp.bfloat16)
            start(0, 0, sub, make_rs(0, 0, sub))
            start(0, 1, sub, make_rs(0, 1, sub))
        xbf[...] = x_ref[...].astype(jnp.bfloat16)
        for s in range(N_DEV - 1):
            recv_slot = (s + 1) % DEPTH
            c_dir = (
                lax.rem(my + 2 * N_DEV - s - 1, N_DEV),
                lax.rem(my + s + 1, N_DEV),
            )
            for sub in range(N_SUB):
                for direction in range(2):
                    descs[(s, direction, sub)].wait_recv()
                    acc(direction, recv_slot, c_dir[direction], sub)
                    if s < N_DEV - 2:
                        start(s + 1, direction, sub,
                              make_rs(s + 1, direction, sub))

        red_slot = (N_DEV - 1) % DEPTH
        own = (lax.rem(my + 1, N_DEV), lax.rem(my + N_DEV - 1, N_DEV))
        for sub in range(N_SUB):
            rows = pl.ds(sub * chh, chh)
            for direction in range(2):
                ge = _gelu(comms[direction][red_slot, rows, :].astype(jnp.float32))
                out_ref[pl.ds(own[direction] * ch + sub * chh, chh),
                        cols[direction]] = ge.astype(jnp.bfloat16)
                start(N_DEV - 1, direction, sub,
                      make_ag(N_DEV - 1, direction, sub, own[direction]))

        for t in range(N_DEV - 1):
            s = N_DEV - 1 + t
            for sub in range(N_SUB):
                for direction in range(2):
                    descs[(s, direction, sub)].wait_recv()
                    if t < N_DEV - 2:
                        c_recv = (
                            lax.rem(my + 2 * N_DEV - t, N_DEV),
                            lax.rem(my + t, N_DEV),
                        )[direction]
                        start(s + 1, direction, sub,
                              make_ag(s + 1, direction, sub, c_recv))

        for s in range(N_HOPS - DEPTH, N_HOPS):
            for sub in range(N_SUB):
                for direction in range(2):
                    descs[(s, direction, sub)].wait_send()

    out_shape = jax.ShapeDtypeStruct((m, n), jnp.bfloat16)
    return pl.pallas_call(
        body,
        out_shape=out_shape,
        in_specs=[
            pl.BlockSpec(memory_space=pltpu.VMEM),
            pl.BlockSpec(memory_space=pltpu.VMEM),
        ],
        out_specs=pl.BlockSpec(memory_space=pltpu.VMEM),
        scratch_shapes=[
            pltpu.VMEM((DEPTH, ch, nh), jnp.bfloat16),
            pltpu.VMEM((DEPTH, ch, nh), jnp.bfloat16),
            pltpu.VMEM((m, k_sh), jnp.bfloat16),
            pltpu.VMEM((k_sh, n), jnp.bfloat16),
            pltpu.SemaphoreType.DMA((2, N_SUB, 2, DEPTH)),
        ],
        compiler_params=pltpu.CompilerParams(
            collective_id=0,
            vmem_limit_bytes=60 * 1024 * 1024,
        ),
    )(x, w_mat)
